# baseline (speedup 1.0000x reference)
import functools

import jax
import jax.numpy as jnp
from jax import lax
from jax.experimental import pallas as pl
from jax.experimental.pallas import tpu as pltpu
from jax.experimental.pallas import tpu_sc as plsc

NG = 256
D = 128
N = 100000
NC = 2
NS = 16
NW = NC * NS
CHUNK = 128
NFULL = N // CHUNK
TAIL = N - NFULL * CHUNK
TAIL_START = NFULL * CHUNK
BASE = NFULL // NW
EXTRA = NFULL - BASE * NW
MAXC = BASE + 1

LANES = 16

_mesh = plsc.VectorSubcoreMesh(
    core_axis_name="c", subcore_axis_name="s", num_cores=NC, num_subcores=NS
)


def _row_add(dst, src, nrows):
  def row(r, carry):
    for j in range(D // LANES):
      sl = pl.ds(j * LANES, LANES)
      dst[r, sl] = dst[r, sl] + src[r, sl]
    return carry

  lax.fori_loop(0, nrows, row, 0)


@functools.partial(
    pl.kernel,
    out_type=jax.ShapeDtypeStruct((NC, NG, D), jnp.float32),
    mesh=_mesh,
    scratch_types=[
        pltpu.VMEM((CHUNK, D), jnp.float32),
        pltpu.VMEM((CHUNK, D), jnp.float32),
        pltpu.VMEM((CHUNK, D), jnp.float32),
        pltpu.VMEM((CHUNK, D), jnp.float32),
        pltpu.VMEM((MAXC, CHUNK), jnp.int32),
        pltpu.VMEM((CHUNK, D), jnp.float32),
        pltpu.VMEM((CHUNK,), jnp.int32),
        pltpu.VMEM((CHUNK,), jnp.int32),
        pltpu.VMEM((TAIL, D), jnp.float32),
        pltpu.VMEM((TAIL,), jnp.int32),
        pltpu.VMEM_SHARED((NG, D), jnp.float32),
        pltpu.SemaphoreType.DMA,
        pltpu.SemaphoreType.DMA,
        pltpu.SemaphoreType.DMA,
        pltpu.SemaphoreType.DMA,
    ],
)
def _segment_sum(h_hbm, idx3_hbm, ci3_hbm, si3_hbm, idxt_hbm, zeros_hbm,
                 out_hbm, hbuf0, hbuf1, hbuf2, hbuf3, ibuf2, srows, cibuf,
                 sfbuf, hbuft, ibuft, acc, sem0, sem1, sem2, sem3):
  c = lax.axis_index("c")
  s = lax.axis_index("s")
  w = c * NS + s
  has_extra = w < EXTRA
  start = BASE * w + jnp.minimum(w, EXTRA)

  @pl.when(s == 0)
  def _():
    pltpu.sync_copy(zeros_hbm, acc)

  plsc.subcore_barrier()

  pltpu.sync_copy(idx3_hbm.at[w], ibuf2)
  pltpu.sync_copy(ci3_hbm.at[w], cibuf)
  pltpu.sync_copy(si3_hbm.at[w], sfbuf)

  def zrow(i, carry):
    for j in range(D // LANES):
      srows[i, pl.ds(j * LANES, LANES)] = jnp.zeros((LANES,), jnp.float32)
    return carry

  lax.fori_loop(0, CHUNK, zrow, 0)

  bufs = (hbuf0, hbuf1, hbuf2, hbuf3)
  sems = (sem0, sem1, sem2, sem3)
  NBUF = 4

  def issue(i):
    b = i % NBUF
    r0 = pl.multiple_of((start + i) * CHUNK, CHUNK)
    pltpu.async_copy(h_hbm.at[pl.ds(r0, CHUNK), :], bufs[b], sems[b])

  def complete(i):
    b = i % NBUF
    pltpu.make_async_copy(
        h_hbm.at[pl.ds(0, CHUNK), :], bufs[b], sems[b]).wait()

    def single_case():
      init = tuple(
          bufs[b][0, pl.ds(j * LANES, LANES)] for j in range(D // LANES))

      def body(r, accv):
        return tuple(
            accv[j] + bufs[b][r, pl.ds(j * LANES, LANES)]
            for j in range(D // LANES))

      accv = lax.fori_loop(1, CHUNK, body, init)
      for j in range(D // LANES):
        srows[i, pl.ds(j * LANES, LANES)] = accv[j]

    def multi_case():
      pltpu.sync_copy(bufs[b], acc.at[ibuf2.at[i]], add=True)

    flag = sfbuf[pl.ds(i, LANES)][0]
    lax.cond(flag == 1, single_case, multi_case)

  for i in range(min(NBUF, BASE)):
    issue(i)
  for i in range(MAXC):
    if i < BASE:
      complete(i)
    else:
      @pl.when(has_extra)
      def _(i=i):
        complete(i)
    nxt = i + NBUF
    if nxt < BASE:
      issue(nxt)
    elif nxt == BASE:
      @pl.when(has_extra)
      def _(i=i):
        issue(i + NBUF)

  pltpu.sync_copy(srows, acc.at[cibuf], add=True)

  @pl.when(w == NW - 1)
  def _():
    pltpu.sync_copy(idxt_hbm, ibuft)
    pltpu.sync_copy(h_hbm.at[pl.ds(TAIL_START, TAIL), :], hbuft)
    pltpu.sync_copy(hbuft, acc.at[ibuft], add=True)

  plsc.subcore_barrier()

  @pl.when(s == 0)
  def _():
    pltpu.sync_copy(acc, out_hbm.at[c])


def _linear_body(p_ref, w_ref, b_ref, y_ref):
  acc = p_ref[0] + p_ref[1]
  y = lax.dot_general(acc, w_ref[...], (((1,), (1,)), ((), ())),
                      preferred_element_type=jnp.float32)
  y_ref[...] = jnp.maximum(y + b_ref[...], 0.0)


@functools.partial(
    pl.kernel,
    out_type=jax.ShapeDtypeStruct((N, D), jnp.float32),
    mesh=_mesh,
    scratch_types=[
        pltpu.VMEM((CHUNK, D), jnp.float32),
        pltpu.VMEM((CHUNK, D), jnp.float32),
        pltpu.VMEM((CHUNK, D), jnp.float32),
        pltpu.VMEM((CHUNK, D), jnp.float32),
        pltpu.VMEM((CHUNK, D), jnp.float32),
        pltpu.VMEM((CHUNK, D), jnp.float32),
        pltpu.VMEM((MAXC, CHUNK), jnp.int32),
        pltpu.VMEM((TAIL, D), jnp.float32),
        pltpu.VMEM((TAIL, D), jnp.float32),
        pltpu.VMEM((TAIL,), jnp.int32),
        pltpu.VMEM_SHARED((NG, D), jnp.float32),
        pltpu.SemaphoreType.DMA,
        pltpu.SemaphoreType.DMA,
        pltpu.SemaphoreType.DMA,
        pltpu.SemaphoreType.DMA,
        pltpu.SemaphoreType.DMA,
        pltpu.SemaphoreType.DMA,
        pltpu.SemaphoreType.DMA,
        pltpu.SemaphoreType.DMA,
        pltpu.SemaphoreType.DMA,
        pltpu.SemaphoreType.DMA,
    ],
)
def _broadcast_add(h_hbm, idx3_hbm, idxt_hbm, y_hbm, out_hbm,
                   hbuf0, hbuf1, hbuf2, ybuf0, ybuf1, ybuf2, ibuf2,
                   hbuft, ybuft, ibuft, ysh,
                   sh0, sh1, sh2, sy0, sy1, sy2, so0, so1, so2, st):
  c = lax.axis_index("c")
  s = lax.axis_index("s")
  w = c * NS + s
  has_extra = w < EXTRA
  start = BASE * w + jnp.minimum(w, EXTRA)

  @pl.when(s == 0)
  def _():
    pltpu.sync_copy(y_hbm, ysh)

  plsc.subcore_barrier()

  pltpu.sync_copy(idx3_hbm.at[w], ibuf2)

  hbufs = (hbuf0, hbuf1, hbuf2)
  ybufs = (ybuf0, ybuf1, ybuf2)
  sems_h = (sh0, sh1, sh2)
  sems_y = (sy0, sy1, sy2)
  sems_o = (so0, so1, so2)
  NB = 3

  def issue(i):
    b = i % NB
    if i >= NB:
      pltpu.make_async_copy(
          hbufs[b], out_hbm.at[pl.ds(0, CHUNK), :], sems_o[b]).wait()
    r0 = pl.multiple_of((start + i) * CHUNK, CHUNK)
    pltpu.async_copy(h_hbm.at[pl.ds(r0, CHUNK), :], hbufs[b], sems_h[b])
    pltpu.async_copy(ysh.at[ibuf2.at[i]], ybufs[b], sems_y[b])

  def complete(i):
    b = i % NB
    pltpu.make_async_copy(
        h_hbm.at[pl.ds(0, CHUNK), :], hbufs[b], sems_h[b]).wait()
    pltpu.make_async_copy(
        ysh.at[ibuf2.at[i]], ybufs[b], sems_y[b]).wait()
    _row_add(hbufs[b], ybufs[b], CHUNK)
    r0 = pl.multiple_of((start + i) * CHUNK, CHUNK)
    pltpu.async_copy(
        hbufs[b], out_hbm.at[pl.ds(r0, CHUNK), :], sems_o[b])

  issue(0)
  for i in range(MAXC):
    nxt = i + 1
    if nxt < BASE:
      issue(nxt)
    elif nxt == BASE:
      @pl.when(has_extra)
      def _(i=i):
        issue(i + 1)
    if i < BASE:
      complete(i)
    else:
      @pl.when(has_extra)
      def _(i=i):
        complete(i)

  for b in range(NB):
    pltpu.make_async_copy(
        hbufs[b], out_hbm.at[pl.ds(0, CHUNK), :], sems_o[b]).wait()

  @pl.when(w == NW - 1)
  def _():
    pltpu.sync_copy(idxt_hbm, ibuft)
    pltpu.sync_copy(h_hbm.at[pl.ds(TAIL_START, TAIL), :], hbuft)
    pltpu.async_copy(ysh.at[ibuft], ybuft, st).wait()
    _row_add(hbuft, ybuft, TAIL)
    pltpu.sync_copy(hbuft, out_hbm.at[pl.ds(TAIL_START, TAIL), :])


@jax.jit
def kernel(H, batch_idx, W, b):
  idx32 = batch_idx.astype(jnp.int32)
  idx2 = jnp.concatenate(
      [idx32[:TAIL_START].reshape(NFULL, CHUNK),
       jnp.zeros((1, CHUNK), jnp.int32)], axis=0)
  starts = [BASE * w + min(w, EXTRA) for w in range(NW)]
  idx3 = jnp.stack([lax.slice_in_dim(idx2, sw, sw + MAXC) for sw in starts])
  first = idx2[:, 0]
  single = (idx2[:, 0] == idx2[:, CHUNK - 1]).astype(jnp.int32)
  pad = jnp.zeros((NW, CHUNK - MAXC), jnp.int32)
  ci3 = jnp.concatenate(
      [jnp.stack([lax.slice_in_dim(first, sw, sw + MAXC) for sw in starts]),
       pad], axis=1)
  si3 = jnp.concatenate(
      [jnp.stack([lax.slice_in_dim(single, sw, sw + MAXC) for sw in starts]),
       pad], axis=1)
  idx_tail = idx32[TAIL_START:]
  zeros = jnp.zeros((NG, D), jnp.float32)
  partials = _segment_sum(H, idx3, ci3, si3, idx_tail, zeros)
  y = pl.pallas_call(
      _linear_body,
      out_shape=jax.ShapeDtypeStruct((NG, D), jnp.float32),
  )(partials, W, b.reshape(1, D))
  return _broadcast_add(H, idx3, idx_tail, y)


# speedup vs baseline: 1.0090x; 1.0090x over previous
import functools

import jax
import jax.numpy as jnp
from jax import lax
from jax.experimental import pallas as pl
from jax.experimental.pallas import tpu as pltpu
from jax.experimental.pallas import tpu_sc as plsc

NG = 256
D = 128
N = 100000
NC = 2
NS = 16
NW = NC * NS
CHUNK = 128
NFULL = N // CHUNK
TAIL = N - NFULL * CHUNK
TAIL_START = NFULL * CHUNK
BASE = NFULL // NW
EXTRA = NFULL - BASE * NW
MAXC = BASE + 1

LANES = 16

_mesh = plsc.VectorSubcoreMesh(
    core_axis_name="c", subcore_axis_name="s", num_cores=NC, num_subcores=NS
)


def _row_add(dst, src, nrows):
  def row(r, carry):
    for j in range(D // LANES):
      sl = pl.ds(j * LANES, LANES)
      dst[r, sl] = dst[r, sl] + src[r, sl]
    return carry

  lax.fori_loop(0, nrows, row, 0)


@functools.partial(
    pl.kernel,
    out_type=jax.ShapeDtypeStruct((NC, NG, D), jnp.float32),
    mesh=_mesh,
    scratch_types=[
        pltpu.VMEM((CHUNK, D), jnp.float32),
        pltpu.VMEM((CHUNK, D), jnp.float32),
        pltpu.VMEM((CHUNK, D), jnp.float32),
        pltpu.VMEM((CHUNK, D), jnp.float32),
        pltpu.VMEM((MAXC, CHUNK), jnp.int32),
        pltpu.VMEM((CHUNK, D), jnp.float32),
        pltpu.VMEM((CHUNK,), jnp.int32),
        pltpu.VMEM((CHUNK,), jnp.int32),
        pltpu.VMEM((TAIL, D), jnp.float32),
        pltpu.VMEM((TAIL,), jnp.int32),
        pltpu.VMEM_SHARED((NG, D), jnp.float32),
        pltpu.SemaphoreType.DMA,
        pltpu.SemaphoreType.DMA,
        pltpu.SemaphoreType.DMA,
        pltpu.SemaphoreType.DMA,
    ],
)
def _segment_sum(h_hbm, idx3_hbm, ci3_hbm, si3_hbm, idxt_hbm, zeros_hbm,
                 out_hbm, hbuf0, hbuf1, hbuf2, hbuf3, ibuf2, srows, cibuf,
                 sfbuf, hbuft, ibuft, acc, sem0, sem1, sem2, sem3):
  c = lax.axis_index("c")
  s = lax.axis_index("s")
  w = c * NS + s
  has_extra = w < EXTRA
  start = BASE * w + jnp.minimum(w, EXTRA)

  @pl.when(s == 0)
  def _():
    pltpu.sync_copy(zeros_hbm, acc)

  plsc.subcore_barrier()

  pltpu.sync_copy(idx3_hbm.at[w], ibuf2)
  pltpu.sync_copy(ci3_hbm.at[w], cibuf)
  pltpu.sync_copy(si3_hbm.at[w], sfbuf)

  def zrow(i, carry):
    for j in range(D // LANES):
      srows[i, pl.ds(j * LANES, LANES)] = jnp.zeros((LANES,), jnp.float32)
    return carry

  lax.fori_loop(0, CHUNK, zrow, 0)

  bufs = (hbuf0, hbuf1, hbuf2, hbuf3)
  sems = (sem0, sem1, sem2, sem3)
  NBUF = 4

  def issue(i):
    b = i % NBUF
    r0 = pl.multiple_of((start + i) * CHUNK, CHUNK)
    pltpu.async_copy(h_hbm.at[pl.ds(r0, CHUNK), :], bufs[b], sems[b])

  def complete(i):
    b = i % NBUF
    pltpu.make_async_copy(
        h_hbm.at[pl.ds(0, CHUNK), :], bufs[b], sems[b]).wait()

    def single_case():
      init = tuple(
          bufs[b][0, pl.ds(j * LANES, LANES)] for j in range(D // LANES))

      def body(r, accv):
        return tuple(
            accv[j] + bufs[b][r, pl.ds(j * LANES, LANES)]
            for j in range(D // LANES))

      accv = lax.fori_loop(1, CHUNK, body, init)
      for j in range(D // LANES):
        srows[i, pl.ds(j * LANES, LANES)] = accv[j]

    def multi_case():
      pltpu.sync_copy(bufs[b], acc.at[ibuf2.at[i]], add=True)

    flag = sfbuf[pl.ds(i, LANES)][0]
    lax.cond(flag == 1, single_case, multi_case)

  for i in range(min(NBUF, BASE)):
    issue(i)
  for i in range(MAXC):
    if i < BASE:
      complete(i)
    else:
      @pl.when(has_extra)
      def _(i=i):
        complete(i)
    nxt = i + NBUF
    if nxt < BASE:
      issue(nxt)
    elif nxt == BASE:
      @pl.when(has_extra)
      def _(i=i):
        issue(i + NBUF)

  pltpu.sync_copy(srows, acc.at[cibuf], add=True)

  @pl.when(w == NW - 1)
  def _():
    pltpu.sync_copy(idxt_hbm, ibuft)
    pltpu.sync_copy(h_hbm.at[pl.ds(TAIL_START, TAIL), :], hbuft)
    pltpu.sync_copy(hbuft, acc.at[ibuft], add=True)

  plsc.subcore_barrier()

  @pl.when(s == 0)
  def _():
    pltpu.sync_copy(acc, out_hbm.at[c])


def _linear_body(p_ref, w_ref, b_ref, y_ref):
  acc = p_ref[0] + p_ref[1]
  y = lax.dot_general(acc, w_ref[...], (((1,), (1,)), ((), ())),
                      preferred_element_type=jnp.float32)
  y_ref[...] = jnp.maximum(y + b_ref[...], 0.0)


@functools.partial(
    pl.kernel,
    out_type=jax.ShapeDtypeStruct((N, D), jnp.float32),
    mesh=_mesh,
    scratch_types=[
        pltpu.VMEM((CHUNK, D), jnp.float32),
        pltpu.VMEM((CHUNK, D), jnp.float32),
        pltpu.VMEM((CHUNK, D), jnp.float32),
        pltpu.VMEM((CHUNK, D), jnp.float32),
        pltpu.VMEM((CHUNK, D), jnp.float32),
        pltpu.VMEM((CHUNK, D), jnp.float32),
        pltpu.VMEM((MAXC, CHUNK), jnp.int32),
        pltpu.VMEM((TAIL, D), jnp.float32),
        pltpu.VMEM((TAIL, D), jnp.float32),
        pltpu.VMEM((TAIL,), jnp.int32),
        pltpu.VMEM_SHARED((NG, D), jnp.float32),
        pltpu.SemaphoreType.DMA,
        pltpu.SemaphoreType.DMA,
        pltpu.SemaphoreType.DMA,
        pltpu.SemaphoreType.DMA,
        pltpu.SemaphoreType.DMA,
        pltpu.SemaphoreType.DMA,
        pltpu.SemaphoreType.DMA,
        pltpu.SemaphoreType.DMA,
        pltpu.SemaphoreType.DMA,
        pltpu.SemaphoreType.DMA,
    ],
)
def _broadcast_add(h_hbm, idx3_hbm, idxt_hbm, y_hbm, out_hbm,
                   hbuf0, hbuf1, hbuf2, ybuf0, ybuf1, ybuf2, ibuf2,
                   hbuft, ybuft, ibuft, ysh,
                   sh0, sh1, sh2, sy0, sy1, sy2, so0, so1, so2, st):
  c = lax.axis_index("c")
  s = lax.axis_index("s")
  w = c * NS + s
  has_extra = w < EXTRA
  start = BASE * w + jnp.minimum(w, EXTRA)

  hbufs = (hbuf0, hbuf1, hbuf2)
  ybufs = (ybuf0, ybuf1, ybuf2)
  sems_h = (sh0, sh1, sh2)
  sems_y = (sy0, sy1, sy2)
  sems_o = (so0, so1, so2)
  NB = 3

  def issue_h(i):
    b = i % NB
    if i >= NB:
      pltpu.make_async_copy(
          hbufs[b], out_hbm.at[pl.ds(0, CHUNK), :], sems_o[b]).wait()
    r0 = pl.multiple_of((start + i) * CHUNK, CHUNK)
    pltpu.async_copy(h_hbm.at[pl.ds(r0, CHUNK), :], hbufs[b], sems_h[b])

  def issue_y(i):
    b = i % NB
    pltpu.async_copy(ysh.at[ibuf2.at[i]], ybufs[b], sems_y[b])

  def issue(i):
    issue_h(i)
    issue_y(i)

  issue_h(0)
  pltpu.sync_copy(idx3_hbm.at[w], ibuf2)

  @pl.when(s == 0)
  def _():
    pltpu.sync_copy(y_hbm, ysh)

  plsc.subcore_barrier()

  issue_y(0)

  def complete(i):
    b = i % NB
    pltpu.make_async_copy(
        h_hbm.at[pl.ds(0, CHUNK), :], hbufs[b], sems_h[b]).wait()
    pltpu.make_async_copy(
        ysh.at[ibuf2.at[i]], ybufs[b], sems_y[b]).wait()
    _row_add(hbufs[b], ybufs[b], CHUNK)
    r0 = pl.multiple_of((start + i) * CHUNK, CHUNK)
    pltpu.async_copy(
        hbufs[b], out_hbm.at[pl.ds(r0, CHUNK), :], sems_o[b])

  for i in range(MAXC):
    nxt = i + 1
    if nxt < BASE:
      issue(nxt)
    elif nxt == BASE:
      @pl.when(has_extra)
      def _(i=i):
        issue(i + 1)
    if i < BASE:
      complete(i)
    else:
      @pl.when(has_extra)
      def _(i=i):
        complete(i)

  for b in range(NB):
    pltpu.make_async_copy(
        hbufs[b], out_hbm.at[pl.ds(0, CHUNK), :], sems_o[b]).wait()

  @pl.when(w == NW - 1)
  def _():
    pltpu.sync_copy(idxt_hbm, ibuft)
    pltpu.sync_copy(h_hbm.at[pl.ds(TAIL_START, TAIL), :], hbuft)
    pltpu.async_copy(ysh.at[ibuft], ybuft, st).wait()
    _row_add(hbuft, ybuft, TAIL)
    pltpu.sync_copy(hbuft, out_hbm.at[pl.ds(TAIL_START, TAIL), :])


@jax.jit
def kernel(H, batch_idx, W, b):
  idx32 = batch_idx.astype(jnp.int32)
  idx2 = jnp.concatenate(
      [idx32[:TAIL_START].reshape(NFULL, CHUNK),
       jnp.zeros((1, CHUNK), jnp.int32)], axis=0)
  starts = [BASE * w + min(w, EXTRA) for w in range(NW)]
  idx3 = jnp.stack([lax.slice_in_dim(idx2, sw, sw + MAXC) for sw in starts])
  first = idx2[:, 0]
  single = (idx2[:, 0] == idx2[:, CHUNK - 1]).astype(jnp.int32)
  pad = jnp.zeros((NW, CHUNK - MAXC), jnp.int32)
  ci3 = jnp.concatenate(
      [jnp.stack([lax.slice_in_dim(first, sw, sw + MAXC) for sw in starts]),
       pad], axis=1)
  si3 = jnp.concatenate(
      [jnp.stack([lax.slice_in_dim(single, sw, sw + MAXC) for sw in starts]),
       pad], axis=1)
  idx_tail = idx32[TAIL_START:]
  zeros = jnp.zeros((NG, D), jnp.float32)
  partials = _segment_sum(H, idx3, ci3, si3, idx_tail, zeros)
  y = pl.pallas_call(
      _linear_body,
      out_shape=jax.ShapeDtypeStruct((NG, D), jnp.float32),
  )(partials, W, b.reshape(1, D))
  return _broadcast_add(H, idx3, idx_tail, y)
